# baseline (speedup 1.0000x reference)
import functools

import jax
import jax.numpy as jnp
from jax import lax
from jax.experimental import pallas as pl
from jax.experimental.pallas import tpu as pltpu
from jax.experimental.pallas import tpu_sc as plsc

HIDDEN = 256
BM = 640

_NC, _NS = 2, 16
_NW = _NC * _NS


def _sc_gather(table, idx, chunk):
    b = idx.shape[0]
    d = table.shape[1]
    dt = table.dtype
    n_chunks = b // chunk
    max_mine = -(-n_chunks // _NW)
    bpad = max_mine * chunk * _NW
    if bpad > b:
        idx = jnp.concatenate(
            [idx, jnp.zeros((bpad - b,), jnp.int32)])
    mesh = plsc.VectorSubcoreMesh(core_axis_name="c", subcore_axis_name="s")

    @functools.partial(
        pl.kernel,
        mesh=mesh,
        out_type=jax.ShapeDtypeStruct((b, d), dt),
        scratch_types=[
            pltpu.VMEM((max_mine * chunk,), jnp.int32),
            pltpu.VMEM((chunk, d), dt),
            pltpu.VMEM((chunk, d), dt),
            pltpu.SemaphoreType.DMA,
        ],
    )
    def k(table_hbm, idx_hbm, out_hbm, idx_all, buf0, buf1, sem):
        wid = lax.axis_index("s") * _NC + lax.axis_index("c")
        base = n_chunks // _NW
        rem = n_chunks % _NW
        n_mine = base + jnp.where(wid < rem, 1, 0)
        start = wid * base + jnp.minimum(wid, rem)

        def _src(i):
            return table_hbm.at[idx_all.at[pl.ds(i * chunk, chunk)]]

        @pl.when(n_mine > 0)
        def _prologue():
            pltpu.sync_copy(idx_hbm.at[pl.ds(start * chunk, max_mine * chunk)],
                            idx_all)
            pltpu.async_copy(_src(0), buf0, sem)

        def body(i, carry):
            def step(cur_b, nxt_b):
                @pl.when(i + 1 < n_mine)
                def _issue():
                    pltpu.async_copy(_src(i + 1), nxt_b, sem)

                pltpu.make_async_copy(_src(i), cur_b, sem).wait()
                off = (start + i) * chunk
                pltpu.sync_copy(cur_b, out_hbm.at[pl.ds(off, chunk)])

            @pl.when(i % 2 == 0)
            def _even():
                step(buf0, buf1)

            @pl.when(i % 2 == 1)
            def _odd():
                step(buf1, buf0)

            return carry

        lax.fori_loop(0, n_mine, body, 0)

    return k(table, idx)


def _pad_rows(x, bm):
    pad = (-x.shape[0]) % bm
    if pad:
        x = jnp.pad(x, ((0, pad),) + ((0, 0),) * (x.ndim - 1))
    return x


def _bdot(x, w):
    return jnp.dot(x.astype(jnp.bfloat16), w.astype(jnp.bfloat16),
                   preferred_element_type=jnp.float32)


def _sc_scatter_add(cat, idx, n_rows):
    b = cat.shape[0]
    C = 128
    W = 128
    n_chunks = b // C
    ZC = 200
    n_zchunks = n_rows // ZC
    assert n_rows % ZC == 0
    zeros_h = jnp.zeros((ZC, W), jnp.float32)
    mesh = plsc.VectorSubcoreMesh(core_axis_name="c", subcore_axis_name="s")

    @functools.partial(
        pl.kernel,
        mesh=mesh,
        out_type=jax.ShapeDtypeStruct((n_rows, 4 * W), jnp.float32),
        scratch_types=[
            pltpu.VMEM((C,), jnp.int32),
            pltpu.VMEM((C,), jnp.int32),
            pltpu.VMEM((C, W), jnp.float32),
            pltpu.VMEM((C, W), jnp.float32),
            pltpu.VMEM((ZC, W), jnp.float32),
            pltpu.VMEM_SHARED((n_rows, W), jnp.float32),
            pltpu.SemaphoreType.DMA,
        ],
    )
    def k(cat_hbm, idx_hbm, z_hbm, out_hbm, idx0, idx1, buf0, buf1, wb_v,
          acc_sh, sem):
        cid = lax.axis_index("c")
        tid = lax.axis_index("s")
        n_mine = (n_chunks - tid + _NS - 1) // _NS
        n_zmine = (n_zchunks - tid + _NS - 1) // _NS
        for s in range(2):
            c0 = s * W

            def zero_body(j, carry):
                rr = (tid + j * _NS) * ZC
                pltpu.sync_copy(z_hbm, acc_sh.at[pl.ds(rr, ZC)])
                return carry

            lax.fori_loop(0, n_zmine, zero_body, 0)
            plsc.subcore_barrier()

            def chunk_body(i, carry):
                off = (tid + i * _NS) * C
                pltpu.sync_copy(idx_hbm.at[pl.ds(off, C)], idx0)
                pltpu.sync_copy(
                    cat_hbm.at[pl.ds(off, C), pl.ds(cid * 2 * W + c0, W)],
                    buf0)
                pltpu.sync_copy(buf0, acc_sh.at[idx0], add=True)
                return carry

            lax.fori_loop(0, n_mine, chunk_body, 0)
            plsc.subcore_barrier()

            def wb_body(j, carry):
                rr = (tid + j * _NS) * ZC
                pltpu.sync_copy(acc_sh.at[pl.ds(rr, ZC)], wb_v)
                pltpu.sync_copy(
                    wb_v, out_hbm.at[pl.ds(rr, ZC), pl.ds(cid * 2 * W + c0, W)])
                return carry

            lax.fori_loop(0, n_zmine, wb_body, 0)
            plsc.subcore_barrier()

    return k(cat, idx, zeros_h)


def _lin_body(x_ref, w_ref, b_ref, o_ref):
    o_ref[:] = _bdot(x_ref[:], w_ref[:]) + b_ref[:]


def _lin_ln_silu_body(x_ref, w_ref, b_ref, g_ref, be_ref, o_ref):
    h = _bdot(x_ref[:], w_ref[:]) + b_ref[:]
    mu = jnp.mean(h, axis=-1, keepdims=True)
    var = jnp.mean((h - mu) ** 2, axis=-1, keepdims=True)
    h = (h - mu) * jax.lax.rsqrt(var + 1e-5) * g_ref[:] + be_ref[:]
    o_ref[:] = h * jax.nn.sigmoid(h)


def _linear(x, W, b, g=None, be=None):
    m0, k = x.shape
    f = W.shape[1]
    xp = _pad_rows(x, BM)
    m = xp.shape[0]
    args = [xp, W, b.reshape(1, f)]
    in_specs = [
        pl.BlockSpec((BM, k), lambda i: (i, 0)),
        pl.BlockSpec((k, f), lambda i: (0, 0)),
        pl.BlockSpec((1, f), lambda i: (0, 0)),
    ]
    if g is not None:
        args += [g.reshape(1, f), be.reshape(1, f)]
        in_specs += [
            pl.BlockSpec((1, f), lambda i: (0, 0)),
            pl.BlockSpec((1, f), lambda i: (0, 0)),
        ]
        body = _lin_ln_silu_body
    else:
        body = _lin_body
    out = pl.pallas_call(
        body,
        grid=(m // BM,),
        in_specs=in_specs,
        out_specs=pl.BlockSpec((BM, f), lambda i: (i, 0)),
        out_shape=jax.ShapeDtypeStruct((m, f), jnp.float32),
    )(*args)
    return out[:m0]


def _mlp(x, p):
    return _linear(x, p["W"], p["b"], p["g"], p["be"])


def _pack16(lo, hi):
    lo_b = lax.bitcast_convert_type(lo.astype(jnp.bfloat16), jnp.uint16)
    hi_b = lax.bitcast_convert_type(hi.astype(jnp.bfloat16), jnp.uint16)
    return lo_b.astype(jnp.uint32) | (hi_b.astype(jnp.uint32) << 16)


def _unpack16(u):
    lo = lax.bitcast_convert_type(u.astype(jnp.uint16), jnp.bfloat16)
    hi = lax.bitcast_convert_type((u >> 16).astype(jnp.uint16), jnp.bfloat16)
    return lo.astype(jnp.float32), hi.astype(jnp.float32)


def _lin3_body(x_ref, w_ref, b_ref, o1_ref, o2_ref, o3_ref):
    h = _bdot(x_ref[:], w_ref[:]) + b_ref[:]
    o1_ref[:] = _pack16(h[:, :HIDDEN], h[:, HIDDEN:2 * HIDDEN])
    o2_ref[:] = _pack16(h[:, 2 * HIDDEN:2 * HIDDEN + 128],
                        h[:, 2 * HIDDEN + 128:3 * HIDDEN])
    o3_ref[:] = h[:, 3 * HIDDEN:]


def _linear3(x, W, b, splits):
    m0, k = x.shape
    f = W.shape[1]
    xp = _pad_rows(x, BM)
    m = xp.shape[0]
    outs = pl.pallas_call(
        _lin3_body,
        grid=(m // BM,),
        in_specs=[
            pl.BlockSpec((BM, k), lambda i: (i, 0)),
            pl.BlockSpec((k, f), lambda i: (0, 0)),
            pl.BlockSpec((1, f), lambda i: (0, 0)),
        ],
        out_specs=[pl.BlockSpec((BM, s), lambda i: (i, 0)) for s in splits],
        out_shape=[
            jax.ShapeDtypeStruct((m, splits[0]), jnp.uint32),
            jax.ShapeDtypeStruct((m, splits[1]), jnp.uint32),
            jax.ShapeDtypeStruct((m, splits[2]), jnp.float32),
        ],
    )(xp, W, b.reshape(1, f))
    return outs


def _edge_body(xsbh_ref, xd_ref, ye_ref, y_ref, g_ref, be_ref,
               cat_ref, ynew_ref):
    xs, bh = _unpack16(xsbh_ref[:])
    xd_lo, xd_hi = _unpack16(xd_ref[:])
    xd = jnp.concatenate([xd_lo, xd_hi], axis=1)
    m = xs + xd + ye_ref[:]
    s = jax.nn.sigmoid(m)
    cat_ref[:, :HIDDEN] = s * bh
    cat_ref[:, HIDDEN:] = s
    mu = jnp.mean(m, axis=-1, keepdims=True)
    var = jnp.mean((m - mu) ** 2, axis=-1, keepdims=True)
    h = (m - mu) * jax.lax.rsqrt(var + 1e-5) * g_ref[:] + be_ref[:]
    ynew_ref[:] = y_ref[:] + h * jax.nn.sigmoid(h)


def _edge_stage(xsbh, xd, ye, y, g, be):
    e = xd.shape[0]
    spec = pl.BlockSpec((BM, HIDDEN), lambda i: (i, 0))
    spec2 = pl.BlockSpec((BM, 2 * HIDDEN), lambda i: (i, 0))
    specp = pl.BlockSpec((BM, HIDDEN), lambda i: (i, 0))
    specph = pl.BlockSpec((BM, HIDDEN // 2), lambda i: (i, 0))
    vspec = pl.BlockSpec((1, HIDDEN), lambda i: (0, 0))
    cat, ynew = pl.pallas_call(
        _edge_body,
        grid=(e // BM,),
        in_specs=[specp, specph, spec, spec, vspec, vspec],
        out_specs=[spec2, spec],
        out_shape=[
            jax.ShapeDtypeStruct((e, 2 * HIDDEN), jnp.float32),
            jax.ShapeDtypeStruct((e, HIDDEN), jnp.float32),
        ],
    )(xsbh, xd, ye, y, g.reshape(1, HIDDEN), be.reshape(1, HIDDEN))
    return cat, ynew


def _node_body(xu_ref, ssh_ref, ss_ref, x_ref, g_ref, be_ref, o_ref):
    h = xu_ref[:] + ssh_ref[:] / (ss_ref[:] + 1e-6)
    mu = jnp.mean(h, axis=-1, keepdims=True)
    var = jnp.mean((h - mu) ** 2, axis=-1, keepdims=True)
    h = (h - mu) * jax.lax.rsqrt(var + 1e-5) * g_ref[:] + be_ref[:]
    o_ref[:] = x_ref[:] + h * jax.nn.sigmoid(h)


def _node_stage(xu, ssh, ss, x, g, be):
    n0 = x.shape[0]
    xu, ssh, ss, x = (_pad_rows(a, BM) for a in (xu, ssh, ss, x))
    n = x.shape[0]
    spec = pl.BlockSpec((BM, HIDDEN), lambda i: (i, 0))
    vspec = pl.BlockSpec((1, HIDDEN), lambda i: (0, 0))
    out = pl.pallas_call(
        _node_body,
        grid=(n // BM,),
        in_specs=[spec, spec, spec, spec, vspec, vspec],
        out_specs=spec,
        out_shape=jax.ShapeDtypeStruct((n, HIDDEN), jnp.float32),
    )(xu, ssh, ss, x, g.reshape(1, HIDDEN), be.reshape(1, HIDDEN))
    return out[:n0]


def _colsum_body(x_ref, o_ref):
    @pl.when(pl.program_id(0) == 0)
    def _init():
        o_ref[:] = jnp.zeros_like(o_ref)

    o_ref[:] += jnp.sum(x_ref[:], axis=0, keepdims=True)


def _colsum(x):
    xp = _pad_rows(x, BM)
    m = xp.shape[0]
    out = pl.pallas_call(
        _colsum_body,
        grid=(m // BM,),
        in_specs=[pl.BlockSpec((BM, HIDDEN), lambda i: (i, 0))],
        out_specs=pl.BlockSpec((1, HIDDEN), lambda i: (0, 0)),
        out_shape=jax.ShapeDtypeStruct((1, HIDDEN), jnp.float32),
    )(xp)
    return out[0]


def _egc(p, src, dst, x, y, n_nodes):
    wcat = jnp.concatenate(
        [p["src_gate"]["W"], p["dst_update"]["W"], p["dst_gate"]["W"],
         p["src_update"]["W"]], axis=1)
    bcat = jnp.concatenate(
        [p["src_gate"]["b"], p["dst_update"]["b"], p["dst_gate"]["b"],
         p["src_update"]["b"]], axis=0)
    xsbh_t, xd_t, xu = _linear3(x, wcat, bcat, (HIDDEN, HIDDEN // 2, HIDDEN))
    ye = _linear(y, p["edge_gate"]["W"], p["edge_gate"]["b"])
    xsbh = _sc_gather(xsbh_t, src, 128)
    xd = _sc_gather(xd_t, dst, 128)
    cat, y_out = _edge_stage(xsbh, xd, ye, y, p["ln_e_g"], p["ln_e_b"])
    if n_nodes * 128 * 4 <= 8 * 1024 * 1024 - 1024:
        sums = _sc_scatter_add(cat, dst, n_nodes)
    else:
        sums = jnp.zeros((n_nodes, 2 * HIDDEN), jnp.float32).at[dst].add(cat)
    x_out = _node_stage(xu[:x.shape[0]], sums[:, :HIDDEN], sums[:, HIDDEN:],
                        x, p["ln_n_g"], p["ln_n_b"])
    return x_out, y_out


def _rbf(d, vmin, vmax, bins):
    centers = jnp.linspace(vmin, vmax, bins)
    gamma = 1.0 / ((vmax - vmin) / (bins - 1))
    return jnp.exp(-gamma * (d[:, None] - centers[None, :]) ** 2)


def kernel(atom_features, r, angle_h, params, edge_index, lg_edge_index):
    n = atom_features.shape[0]
    e = r.shape[0]
    src, dst = edge_index[0], edge_index[1]
    lsrc, ldst = lg_edge_index[0], lg_edge_index[1]

    z = _mlp(_mlp(_rbf(angle_h, -1.0, 1.0, 40), params["angle_emb"]["m1"]),
             params["angle_emb"]["m2"])
    x = _mlp(atom_features, params["atom_emb"])
    bondlength = jnp.linalg.norm(r, axis=1)
    y = _mlp(_mlp(_rbf(bondlength, 0.0, 8.0, 16), params["edge_emb"]["m1"]),
             params["edge_emb"]["m2"])

    for lp in params["alignn"]:
        x, m = _egc(lp["node"], src, dst, x, y, n)
        y, z = _egc(lp["edge"], lsrc, ldst, m, z, e)
    for gp in params["gcn"]:
        x, y = _egc(gp, src, dst, x, y, n)

    h = _colsum(x) / n
    out = h @ params["fc"]["W"] + params["fc"]["b"]
    return jnp.squeeze(out)


# speedup vs baseline: 1.1065x; 1.1065x over previous
import functools

import jax
import jax.numpy as jnp
from jax import lax
from jax.experimental import pallas as pl
from jax.experimental.pallas import tpu as pltpu
from jax.experimental.pallas import tpu_sc as plsc

HIDDEN = 256
BM = 640

_NC, _NS = 2, 16
_NW = _NC * _NS


def _sc_gather(table, idx, chunk):
    b = idx.shape[0]
    d = table.shape[1]
    dt = table.dtype
    n_chunks = b // chunk
    max_mine = -(-n_chunks // _NW)
    bpad = max_mine * chunk * _NW
    if bpad > b:
        idx = jnp.concatenate(
            [idx, jnp.zeros((bpad - b,), jnp.int32)])
    mesh = plsc.VectorSubcoreMesh(core_axis_name="c", subcore_axis_name="s")

    @functools.partial(
        pl.kernel,
        mesh=mesh,
        out_type=jax.ShapeDtypeStruct((b, d), dt),
        scratch_types=[
            pltpu.VMEM((max_mine * chunk,), jnp.int32),
            pltpu.VMEM((chunk, d), dt),
            pltpu.VMEM((chunk, d), dt),
            pltpu.SemaphoreType.DMA,
        ],
    )
    def k(table_hbm, idx_hbm, out_hbm, idx_all, buf0, buf1, sem):
        wid = lax.axis_index("s") * _NC + lax.axis_index("c")
        base = n_chunks // _NW
        rem = n_chunks % _NW
        n_mine = base + jnp.where(wid < rem, 1, 0)
        start = wid * base + jnp.minimum(wid, rem)

        def _src(i):
            return table_hbm.at[idx_all.at[pl.ds(i * chunk, chunk)]]

        @pl.when(n_mine > 0)
        def _prologue():
            pltpu.sync_copy(idx_hbm.at[pl.ds(start * chunk, max_mine * chunk)],
                            idx_all)
            pltpu.async_copy(_src(0), buf0, sem)

        def body(i, carry):
            def step(cur_b, nxt_b):
                @pl.when(i + 1 < n_mine)
                def _issue():
                    pltpu.async_copy(_src(i + 1), nxt_b, sem)

                pltpu.make_async_copy(_src(i), cur_b, sem).wait()
                off = (start + i) * chunk
                pltpu.sync_copy(cur_b, out_hbm.at[pl.ds(off, chunk)])

            @pl.when(i % 2 == 0)
            def _even():
                step(buf0, buf1)

            @pl.when(i % 2 == 1)
            def _odd():
                step(buf1, buf0)

            return carry

        lax.fori_loop(0, n_mine, body, 0)

    return k(table, idx)


def _pad_rows(x, bm):
    pad = (-x.shape[0]) % bm
    if pad:
        x = jnp.pad(x, ((0, pad),) + ((0, 0),) * (x.ndim - 1))
    return x


def _bdot(x, w):
    return jnp.dot(x.astype(jnp.bfloat16), w.astype(jnp.bfloat16),
                   preferred_element_type=jnp.float32)


def _sc_scatter_add(cat, idx, n_rows):
    b = cat.shape[0]
    C = 128
    W = 128
    n_chunks = b // C
    ZC = 200
    n_zchunks = n_rows // ZC
    assert n_rows % ZC == 0
    zeros_h = jnp.zeros((ZC, W), jnp.float32)
    mesh = plsc.VectorSubcoreMesh(core_axis_name="c", subcore_axis_name="s")

    @functools.partial(
        pl.kernel,
        mesh=mesh,
        out_type=jax.ShapeDtypeStruct((n_rows, 4 * W), jnp.float32),
        scratch_types=[
            pltpu.VMEM((C,), jnp.int32),
            pltpu.VMEM((C,), jnp.int32),
            pltpu.VMEM((C, W), jnp.float32),
            pltpu.VMEM((C, W), jnp.float32),
            pltpu.VMEM((ZC, W), jnp.float32),
            pltpu.VMEM_SHARED((n_rows, W), jnp.float32),
            pltpu.SemaphoreType.DMA,
        ],
    )
    def k(cat_hbm, idx_hbm, z_hbm, out_hbm, idx0, idx1, buf0, buf1, wb_v,
          acc_sh, sem):
        cid = lax.axis_index("c")
        tid = lax.axis_index("s")
        n_mine = (n_chunks - tid + _NS - 1) // _NS
        n_zmine = (n_zchunks - tid + _NS - 1) // _NS
        for s in range(2):
            c0 = s * W

            def zero_body(j, carry):
                rr = (tid + j * _NS) * ZC
                pltpu.sync_copy(z_hbm, acc_sh.at[pl.ds(rr, ZC)])
                return carry

            lax.fori_loop(0, n_zmine, zero_body, 0)
            plsc.subcore_barrier()

            def chunk_body(i, carry):
                off = (tid + i * _NS) * C
                pltpu.sync_copy(idx_hbm.at[pl.ds(off, C)], idx0)
                pltpu.sync_copy(
                    cat_hbm.at[pl.ds(off, C), pl.ds(cid * 2 * W + c0, W)],
                    buf0)
                pltpu.sync_copy(buf0, acc_sh.at[idx0], add=True)
                return carry

            lax.fori_loop(0, n_mine, chunk_body, 0)
            plsc.subcore_barrier()

            def wb_body(j, carry):
                rr = (tid + j * _NS) * ZC
                pltpu.sync_copy(acc_sh.at[pl.ds(rr, ZC)], wb_v)
                pltpu.sync_copy(
                    wb_v, out_hbm.at[pl.ds(rr, ZC), pl.ds(cid * 2 * W + c0, W)])
                return carry

            lax.fori_loop(0, n_zmine, wb_body, 0)
            plsc.subcore_barrier()

    return k(cat, idx, zeros_h)


def _lin_body(x_ref, w_ref, b_ref, o_ref):
    o_ref[:] = _bdot(x_ref[:], w_ref[:]) + b_ref[:]


def _lin_ln_silu_body(x_ref, w_ref, b_ref, g_ref, be_ref, o_ref):
    h = _bdot(x_ref[:], w_ref[:]) + b_ref[:]
    mu = jnp.mean(h, axis=-1, keepdims=True)
    var = jnp.mean((h - mu) ** 2, axis=-1, keepdims=True)
    h = (h - mu) * jax.lax.rsqrt(var + 1e-5) * g_ref[:] + be_ref[:]
    o_ref[:] = h * jax.nn.sigmoid(h)


def _linear(x, W, b, g=None, be=None):
    m0, k = x.shape
    f = W.shape[1]
    xp = _pad_rows(x, BM)
    m = xp.shape[0]
    args = [xp, W, b.reshape(1, f)]
    in_specs = [
        pl.BlockSpec((BM, k), lambda i: (i, 0)),
        pl.BlockSpec((k, f), lambda i: (0, 0)),
        pl.BlockSpec((1, f), lambda i: (0, 0)),
    ]
    if g is not None:
        args += [g.reshape(1, f), be.reshape(1, f)]
        in_specs += [
            pl.BlockSpec((1, f), lambda i: (0, 0)),
            pl.BlockSpec((1, f), lambda i: (0, 0)),
        ]
        body = _lin_ln_silu_body
    else:
        body = _lin_body
    out = pl.pallas_call(
        body,
        grid=(m // BM,),
        in_specs=in_specs,
        out_specs=pl.BlockSpec((BM, f), lambda i: (i, 0)),
        out_shape=jax.ShapeDtypeStruct((m, f), jnp.float32),
    )(*args)
    return out[:m0]


def _mlp(x, p):
    return _linear(x, p["W"], p["b"], p["g"], p["be"])


def _pack16(lo, hi):
    lo_b = lax.bitcast_convert_type(lo.astype(jnp.bfloat16), jnp.uint16)
    hi_b = lax.bitcast_convert_type(hi.astype(jnp.bfloat16), jnp.uint16)
    return lo_b.astype(jnp.uint32) | (hi_b.astype(jnp.uint32) << 16)


def _unpack16(u):
    lo = lax.bitcast_convert_type(u.astype(jnp.uint16), jnp.bfloat16)
    hi = lax.bitcast_convert_type((u >> 16).astype(jnp.uint16), jnp.bfloat16)
    return lo.astype(jnp.float32), hi.astype(jnp.float32)


def _lin3_body(x_ref, w_ref, b_ref, o1_ref, o2_ref, o3_ref):
    h = _bdot(x_ref[:], w_ref[:]) + b_ref[:]
    o1_ref[:] = _pack16(h[:, :HIDDEN], h[:, HIDDEN:2 * HIDDEN])
    o2_ref[:] = _pack16(h[:, 2 * HIDDEN:2 * HIDDEN + 128],
                        h[:, 2 * HIDDEN + 128:3 * HIDDEN])
    o3_ref[:] = h[:, 3 * HIDDEN:]


def _linear3(x, W, b, splits):
    m0, k = x.shape
    f = W.shape[1]
    xp = _pad_rows(x, BM)
    m = xp.shape[0]
    outs = pl.pallas_call(
        _lin3_body,
        grid=(m // BM,),
        in_specs=[
            pl.BlockSpec((BM, k), lambda i: (i, 0)),
            pl.BlockSpec((k, f), lambda i: (0, 0)),
            pl.BlockSpec((1, f), lambda i: (0, 0)),
        ],
        out_specs=[pl.BlockSpec((BM, s), lambda i: (i, 0)) for s in splits],
        out_shape=[
            jax.ShapeDtypeStruct((m, splits[0]), jnp.uint32),
            jax.ShapeDtypeStruct((m, splits[1]), jnp.uint32),
            jax.ShapeDtypeStruct((m, splits[2]), jnp.float32),
        ],
    )(xp, W, b.reshape(1, f))
    return outs


def _edge_body(xsbh_ref, xd_ref, y_ref, we_ref, bw_ref, g_ref, be_ref,
               cat_ref, ynew_ref):
    xs, bh = _unpack16(xsbh_ref[:])
    xd_lo, xd_hi = _unpack16(xd_ref[:])
    xd = jnp.concatenate([xd_lo, xd_hi], axis=1)
    ye = _bdot(y_ref[:], we_ref[:]) + bw_ref[:]
    m = xs + xd + ye
    s = jax.nn.sigmoid(m)
    cat_ref[:, :HIDDEN] = s * bh
    cat_ref[:, HIDDEN:] = s
    mu = jnp.mean(m, axis=-1, keepdims=True)
    var = jnp.mean((m - mu) ** 2, axis=-1, keepdims=True)
    h = (m - mu) * jax.lax.rsqrt(var + 1e-5) * g_ref[:] + be_ref[:]
    ynew_ref[:] = y_ref[:] + h * jax.nn.sigmoid(h)


def _edge_stage(xsbh, xd, y, we, bw, g, be):
    e = xd.shape[0]
    spec = pl.BlockSpec((BM, HIDDEN), lambda i: (i, 0))
    spec2 = pl.BlockSpec((BM, 2 * HIDDEN), lambda i: (i, 0))
    specp = pl.BlockSpec((BM, HIDDEN), lambda i: (i, 0))
    specph = pl.BlockSpec((BM, HIDDEN // 2), lambda i: (i, 0))
    wspec = pl.BlockSpec((HIDDEN, HIDDEN), lambda i: (0, 0))
    vspec = pl.BlockSpec((1, HIDDEN), lambda i: (0, 0))
    cat, ynew = pl.pallas_call(
        _edge_body,
        grid=(e // BM,),
        in_specs=[specp, specph, spec, wspec, vspec, vspec, vspec],
        out_specs=[spec2, spec],
        out_shape=[
            jax.ShapeDtypeStruct((e, 2 * HIDDEN), jnp.float32),
            jax.ShapeDtypeStruct((e, HIDDEN), jnp.float32),
        ],
    )(xsbh, xd, y, we, bw.reshape(1, HIDDEN), g.reshape(1, HIDDEN),
      be.reshape(1, HIDDEN))
    return cat, ynew


def _node_body(xu_ref, ssh_ref, ss_ref, x_ref, g_ref, be_ref, o_ref):
    h = xu_ref[:] + ssh_ref[:] / (ss_ref[:] + 1e-6)
    mu = jnp.mean(h, axis=-1, keepdims=True)
    var = jnp.mean((h - mu) ** 2, axis=-1, keepdims=True)
    h = (h - mu) * jax.lax.rsqrt(var + 1e-5) * g_ref[:] + be_ref[:]
    o_ref[:] = x_ref[:] + h * jax.nn.sigmoid(h)


def _node_stage(xu, ssh, ss, x, g, be):
    n0 = x.shape[0]
    xu, ssh, ss, x = (_pad_rows(a, BM) for a in (xu, ssh, ss, x))
    n = x.shape[0]
    spec = pl.BlockSpec((BM, HIDDEN), lambda i: (i, 0))
    vspec = pl.BlockSpec((1, HIDDEN), lambda i: (0, 0))
    out = pl.pallas_call(
        _node_body,
        grid=(n // BM,),
        in_specs=[spec, spec, spec, spec, vspec, vspec],
        out_specs=spec,
        out_shape=jax.ShapeDtypeStruct((n, HIDDEN), jnp.float32),
    )(xu, ssh, ss, x, g.reshape(1, HIDDEN), be.reshape(1, HIDDEN))
    return out[:n0]


def _colsum_body(x_ref, o_ref):
    @pl.when(pl.program_id(0) == 0)
    def _init():
        o_ref[:] = jnp.zeros_like(o_ref)

    o_ref[:] += jnp.sum(x_ref[:], axis=0, keepdims=True)


def _colsum(x):
    xp = _pad_rows(x, BM)
    m = xp.shape[0]
    out = pl.pallas_call(
        _colsum_body,
        grid=(m // BM,),
        in_specs=[pl.BlockSpec((BM, HIDDEN), lambda i: (i, 0))],
        out_specs=pl.BlockSpec((1, HIDDEN), lambda i: (0, 0)),
        out_shape=jax.ShapeDtypeStruct((1, HIDDEN), jnp.float32),
    )(xp)
    return out[0]


def _egc(p, src, dst, x, y, n_nodes):
    wcat = jnp.concatenate(
        [p["src_gate"]["W"], p["dst_update"]["W"], p["dst_gate"]["W"],
         p["src_update"]["W"]], axis=1)
    bcat = jnp.concatenate(
        [p["src_gate"]["b"], p["dst_update"]["b"], p["dst_gate"]["b"],
         p["src_update"]["b"]], axis=0)
    xsbh_t, xd_t, xu = _linear3(x, wcat, bcat, (HIDDEN, HIDDEN // 2, HIDDEN))
    xsbh = _sc_gather(xsbh_t, src, 128)
    xd = _sc_gather(xd_t, dst, 128)
    cat, y_out = _edge_stage(xsbh, xd, y, p["edge_gate"]["W"],
                             p["edge_gate"]["b"], p["ln_e_g"], p["ln_e_b"])
    if n_nodes * 128 * 4 <= 8 * 1024 * 1024 - 1024:
        sums = _sc_scatter_add(cat, dst, n_nodes)
    else:
        sums = jnp.zeros((n_nodes, 2 * HIDDEN), jnp.float32).at[dst].add(cat)
    x_out = _node_stage(xu[:x.shape[0]], sums[:, :HIDDEN], sums[:, HIDDEN:],
                        x, p["ln_n_g"], p["ln_n_b"])
    return x_out, y_out


def _rbf(d, vmin, vmax, bins):
    centers = jnp.linspace(vmin, vmax, bins)
    gamma = 1.0 / ((vmax - vmin) / (bins - 1))
    return jnp.exp(-gamma * (d[:, None] - centers[None, :]) ** 2)


def kernel(atom_features, r, angle_h, params, edge_index, lg_edge_index):
    n = atom_features.shape[0]
    e = r.shape[0]
    src, dst = edge_index[0], edge_index[1]
    lsrc, ldst = lg_edge_index[0], lg_edge_index[1]

    z = _mlp(_mlp(_rbf(angle_h, -1.0, 1.0, 40), params["angle_emb"]["m1"]),
             params["angle_emb"]["m2"])
    x = _mlp(atom_features, params["atom_emb"])
    bondlength = jnp.linalg.norm(r, axis=1)
    y = _mlp(_mlp(_rbf(bondlength, 0.0, 8.0, 16), params["edge_emb"]["m1"]),
             params["edge_emb"]["m2"])

    for lp in params["alignn"]:
        x, m = _egc(lp["node"], src, dst, x, y, n)
        y, z = _egc(lp["edge"], lsrc, ldst, m, z, e)
    for gp in params["gcn"]:
        x, y = _egc(gp, src, dst, x, y, n)

    h = _colsum(x) / n
    out = h @ params["fc"]["W"] + params["fc"]["b"]
    return jnp.squeeze(out)
